# Initial kernel scaffold; baseline (speedup 1.0000x reference)
import jax
import jax.numpy as jnp
from jax import lax
from jax.experimental import pallas as pl
from jax.experimental.pallas import tpu as pltpu
from jax.experimental.pallas import tpu_sc as plsc

F32 = jnp.float32
I32 = jnp.int32


def _row_split(n):
  r = (-(-n // 16) + 7) // 8 * 8
  last = n - 15 * r
  assert last > 0 and r % 8 == 0
  return r, last


def _edge_lstm(eaT, W, be):
  E = eaT.shape[1]
  Eb = 3200 if E % 3200 == 0 else 128
  assert E % Eb == 0

  def body(ea_ref, w_ref, b_ref, h_ref, c_ref):
    g = jnp.dot(w_ref[...], ea_ref[...], preferred_element_type=F32)
    g = g + b_ref[...]
    i = jax.nn.sigmoid(g[0:8])
    gg = jnp.tanh(g[16:24])
    o = jax.nn.sigmoid(g[24:32])
    c2 = i * gg
    c_ref[...] = c2
    h_ref[...] = o * jnp.tanh(c2)

  return pl.pallas_call(
      body,
      grid=(E // Eb,),
      in_specs=[
          pl.BlockSpec((4, Eb), lambda i: (0, i)),
          pl.BlockSpec((32, 4), lambda i: (0, 0)),
          pl.BlockSpec((32, 1), lambda i: (0, 0)),
      ],
      out_specs=[
          pl.BlockSpec((8, Eb), lambda i: (0, i)),
          pl.BlockSpec((8, Eb), lambda i: (0, i)),
      ],
      out_shape=[
          jax.ShapeDtypeStruct((8, E), F32),
          jax.ShapeDtypeStruct((8, E), F32),
      ],
  )(eaT, W, be)


def _p1(h_e2, dst, zs, zc, ones2):
  E = h_e2.shape[0]
  N = zs.shape[0]
  assert E % 32 == 0
  pw = E // 32
  nbf, tail = pw // 128, pw % 128
  assert tail > 0 and pw % 8 == 0
  rA, rB = _row_split(N)
  mesh = plsc.VectorSubcoreMesh(core_axis_name="c", subcore_axis_name="s")

  def body(he, dstn, zs_h, zc_h, ones_h, sums_out, cnt_out,
           idx, idxt, rows, rowst, ones_v, sums_sp, cnt_sp):
    c = lax.axis_index("c")
    s = lax.axis_index("s")
    w = s * 2 + c
    r0 = s * rA

    @pl.when(s < 15)
    def _():
      pltpu.sync_copy(zs_h.at[pl.ds(r0, rA)], sums_sp.at[pl.ds(r0, rA)])
      pltpu.sync_copy(zc_h.at[pl.ds(r0, rA)], cnt_sp.at[pl.ds(r0, rA)])

    @pl.when(s == 15)
    def _():
      pltpu.sync_copy(zs_h.at[pl.ds(15 * rA, rB)], sums_sp.at[pl.ds(15 * rA, rB)])
      pltpu.sync_copy(zc_h.at[pl.ds(15 * rA, rB)], cnt_sp.at[pl.ds(15 * rA, rB)])

    pltpu.sync_copy(ones_h, ones_v)
    plsc.subcore_barrier()

    base = w * pw

    def bb(b, cr):
      off = base + b * 128
      pltpu.sync_copy(dstn.at[pl.ds(off, 128)], idx.at[0])
      pltpu.sync_copy(he.at[pl.ds(off, 128)], rows)
      pltpu.sync_copy(rows, sums_sp.at[idx.at[0]], add=True)
      pltpu.sync_copy(ones_v, cnt_sp.at[idx.at[0]], add=True)
      return cr

    lax.fori_loop(0, nbf, bb, 0)

    offt = base + nbf * 128
    pltpu.sync_copy(dstn.at[pl.ds(offt, tail)], idxt.at[0])
    pltpu.sync_copy(he.at[pl.ds(offt, tail)], rowst)
    pltpu.sync_copy(rowst, sums_sp.at[idxt.at[0]], add=True)
    pltpu.sync_copy(ones_v.at[pl.ds(0, tail)], cnt_sp.at[idxt.at[0]], add=True)

    plsc.subcore_barrier()
    o0 = c * N + r0

    @pl.when(s < 15)
    def _():
      pltpu.sync_copy(sums_sp.at[pl.ds(r0, rA)], sums_out.at[pl.ds(o0, rA)])
      pltpu.sync_copy(cnt_sp.at[pl.ds(r0, rA)], cnt_out.at[pl.ds(o0, rA)])

    @pl.when(s == 15)
    def _():
      pltpu.sync_copy(sums_sp.at[pl.ds(15 * rA, rB)],
                      sums_out.at[pl.ds(c * N + 15 * rA, rB)])
      pltpu.sync_copy(cnt_sp.at[pl.ds(15 * rA, rB)],
                      cnt_out.at[pl.ds(c * N + 15 * rA, rB)])

  kfn = pl.kernel(
      body,
      out_type=[
          jax.ShapeDtypeStruct((2 * N, 8), F32),
          jax.ShapeDtypeStruct((2 * N, 2), F32),
      ],
      mesh=mesh,
      scratch_types=[
          pltpu.VMEM((1, 128), I32),
          pltpu.VMEM((1, tail), I32),
          pltpu.VMEM((128, 8), F32),
          pltpu.VMEM((tail, 8), F32),
          pltpu.VMEM((128, 2), F32),
          pltpu.VMEM_SHARED((N, 8), F32),
          pltpu.VMEM_SHARED((N, 2), F32),
      ],
  )
  return kfn(h_e2, dst, zs, zc, ones2)


def _node_proj(x, WnT, bn, s0, s1, c0, c1, WgT, As, Ad):
  N = x.shape[0]
  Nb = 1000 if N % 1000 == 0 else 8
  assert N % Nb == 0

  def body(x_ref, wn_ref, bn_ref, s0_ref, s1_ref, c0_ref, c1_ref,
           wg_ref, as_ref, ad_ref,
           hn_ref, cn_ref, xl2_ref, atab_ref, exs_ref):
    g = jnp.dot(x_ref[...], wn_ref[...], preferred_element_type=F32)
    g = g + bn_ref[...]
    i = jax.nn.sigmoid(g[:, 0:64])
    gg = jnp.tanh(g[:, 128:192])
    o = jax.nn.sigmoid(g[:, 192:256])
    c2 = i * gg
    h2 = o * jnp.tanh(c2)
    hn_ref[...] = h2
    cn_ref[...] = c2
    sums = s0_ref[...] + s1_ref[...]
    cnt = c0_ref[...] + c1_ref[...]
    ee = sums / jnp.maximum(cnt, 1.0)
    oc = jnp.concatenate([h2, ee], axis=1)
    xl = jnp.dot(oc, wg_ref[...], preferred_element_type=F32)
    a_s = jnp.dot(xl, as_ref[...], preferred_element_type=F32)
    a_d = jnp.dot(xl, ad_ref[...], preferred_element_type=F32)
    al = a_s + a_d
    al = jnp.where(al >= 0, al, 0.2 * al)
    exs_ref[...] = jnp.exp(al)
    xl2_ref[0] = xl[:, 0:32]
    xl2_ref[1] = xl[:, 32:64]
    atab_ref[0] = jnp.concatenate([a_s[:, 0:2], a_d[:, 0:2]], axis=1)
    atab_ref[1] = jnp.concatenate([a_s[:, 2:4], a_d[:, 2:4]], axis=1)

  return pl.pallas_call(
      body,
      grid=(N // Nb,),
      in_specs=[
          pl.BlockSpec((Nb, 128), lambda i: (i, 0)),
          pl.BlockSpec((128, 256), lambda i: (0, 0)),
          pl.BlockSpec((1, 256), lambda i: (0, 0)),
          pl.BlockSpec((Nb, 8), lambda i: (i, 0)),
          pl.BlockSpec((Nb, 8), lambda i: (i, 0)),
          pl.BlockSpec((Nb, 1), lambda i: (i, 0)),
          pl.BlockSpec((Nb, 1), lambda i: (i, 0)),
          pl.BlockSpec((72, 64), lambda i: (0, 0)),
          pl.BlockSpec((64, 4), lambda i: (0, 0)),
          pl.BlockSpec((64, 4), lambda i: (0, 0)),
      ],
      out_specs=[
          pl.BlockSpec((Nb, 64), lambda i: (i, 0)),
          pl.BlockSpec((Nb, 64), lambda i: (i, 0)),
          pl.BlockSpec((2, Nb, 32), lambda i: (0, i, 0)),
          pl.BlockSpec((2, Nb, 4), lambda i: (0, i, 0)),
          pl.BlockSpec((Nb, 4), lambda i: (i, 0)),
      ],
      out_shape=[
          jax.ShapeDtypeStruct((N, 64), F32),
          jax.ShapeDtypeStruct((N, 64), F32),
          jax.ShapeDtypeStruct((2, N, 32), F32),
          jax.ShapeDtypeStruct((2, N, 4), F32),
          jax.ShapeDtypeStruct((N, 4), F32),
      ],
  )(x, WnT, bn, s0, s1, c0, c1, WgT, As, Ad)


def _p2(src, dst, xlf, af, zacc, zden):
  E = src.shape[0]
  N = zacc.shape[0]
  assert E % 16 == 0
  pt = E // 16
  nbf, tail = pt // 128, pt % 128
  assert tail > 0 and tail % 16 == 0 and pt % 8 == 0
  ng_tail = tail // 16
  rA, rB = _row_split(N)
  mesh = plsc.VectorSubcoreMesh(core_axis_name="c", subcore_axis_name="s")

  def body(srcn, dstn, xl_h, a_h, zacc_h, zden_h, acc_out, den_out,
           si, di, sa, da, sit, dit, sat, dat,
           xlr, xlrt, asr, adr, asrt, adrt, ex0, ex1, exd,
           acc_sp, den_sp):
    c = lax.axis_index("c")
    s = lax.axis_index("s")
    cN = c * N
    r0 = s * rA

    @pl.when(s < 15)
    def _():
      pltpu.sync_copy(zacc_h.at[pl.ds(r0, rA)], acc_sp.at[pl.ds(r0, rA)])
      pltpu.sync_copy(zden_h.at[pl.ds(r0, rA)], den_sp.at[pl.ds(r0, rA)])

    @pl.when(s == 15)
    def _():
      pltpu.sync_copy(zacc_h.at[pl.ds(15 * rA, rB)], acc_sp.at[pl.ds(15 * rA, rB)])
      pltpu.sync_copy(zden_h.at[pl.ds(15 * rA, rB)], den_sp.at[pl.ds(15 * rA, rB)])

    plsc.subcore_barrier()

    lanes = lax.iota(I32, 16)
    col0 = jnp.zeros((16,), I32)
    col1 = col0 + 1
    col2 = col0 + 2
    col3 = col0 + 3

    def adjust(raw_ref, adj_ref, n):
      for g in range(n // 16):
        v = raw_ref[0, pl.ds(g * 16, 16)]
        adj_ref[0, pl.ds(g * 16, 16)] = v + cN

    def compute_ex(asr_ref, adr_ref, n_groups):
      for g in range(n_groups):
        ridx = lanes + g * 16
        as0 = plsc.load_gather(asr_ref, [ridx, col0])
        as1 = plsc.load_gather(asr_ref, [ridx, col1])
        ad0 = plsc.load_gather(adr_ref, [ridx, col2])
        ad1 = plsc.load_gather(adr_ref, [ridx, col3])
        a0 = as0 + ad0
        a0 = jnp.where(a0 >= 0, a0, 0.2 * a0)
        e0 = jnp.exp(a0)
        a1 = as1 + ad1
        a1 = jnp.where(a1 >= 0, a1, 0.2 * a1)
        e1 = jnp.exp(a1)
        ex0[pl.ds(g * 16, 16)] = e0
        ex1[pl.ds(g * 16, 16)] = e1
        plsc.store_scatter(exd, [ridx, col0], e0)
        plsc.store_scatter(exd, [ridx, col1], e1)

    def scale(xlr_ref, nb):
      def sb(e, cr):
        s0 = ex0[e]
        s1 = ex1[e]
        xlr_ref[e, pl.ds(0, 16)] = xlr_ref[e, pl.ds(0, 16)] * s0
        xlr_ref[e, pl.ds(16, 16)] = xlr_ref[e, pl.ds(16, 16)] * s1
        return cr
      lax.fori_loop(0, nb, sb, 0)

    base = s * pt

    def bb(b, cr):
      off = base + b * 128
      pltpu.sync_copy(srcn.at[pl.ds(off, 128)], si.at[0])
      pltpu.sync_copy(dstn.at[pl.ds(off, 128)], di.at[0])
      adjust(si, sa, 128)
      adjust(di, da, 128)
      pltpu.sync_copy(xl_h.at[sa.at[0]], xlr)
      pltpu.sync_copy(a_h.at[sa.at[0]], asr)
      pltpu.sync_copy(a_h.at[da.at[0]], adr)
      compute_ex(asr, adr, 8)
      scale(xlr, 128)
      pltpu.sync_copy(xlr, acc_sp.at[di.at[0]], add=True)
      pltpu.sync_copy(exd, den_sp.at[di.at[0]], add=True)
      return cr

    lax.fori_loop(0, nbf, bb, 0)

    offt = base + nbf * 128
    pltpu.sync_copy(srcn.at[pl.ds(offt, tail)], sit.at[0])
    pltpu.sync_copy(dstn.at[pl.ds(offt, tail)], dit.at[0])
    adjust(sit, sat, tail)
    adjust(dit, dat, tail)
    pltpu.sync_copy(xl_h.at[sat.at[0]], xlrt)
    pltpu.sync_copy(a_h.at[sat.at[0]], asrt)
    pltpu.sync_copy(a_h.at[dat.at[0]], adrt)
    compute_ex(asrt, adrt, ng_tail)
    scale(xlrt, tail)
    pltpu.sync_copy(xlrt, acc_sp.at[dit.at[0]], add=True)
    pltpu.sync_copy(exd.at[pl.ds(0, tail)], den_sp.at[dit.at[0]], add=True)

    plsc.subcore_barrier()
    o0 = cN + r0

    @pl.when(s < 15)
    def _():
      pltpu.sync_copy(acc_sp.at[pl.ds(r0, rA)], acc_out.at[pl.ds(o0, rA)])
      pltpu.sync_copy(den_sp.at[pl.ds(r0, rA)], den_out.at[pl.ds(o0, rA)])

    @pl.when(s == 15)
    def _():
      pltpu.sync_copy(acc_sp.at[pl.ds(15 * rA, rB)],
                      acc_out.at[pl.ds(cN + 15 * rA, rB)])
      pltpu.sync_copy(den_sp.at[pl.ds(15 * rA, rB)],
                      den_out.at[pl.ds(cN + 15 * rA, rB)])

  kfn = pl.kernel(
      body,
      out_type=[
          jax.ShapeDtypeStruct((2 * N, 32), F32),
          jax.ShapeDtypeStruct((2 * N, 2), F32),
      ],
      mesh=mesh,
      scratch_types=[
          pltpu.VMEM((1, 128), I32),
          pltpu.VMEM((1, 128), I32),
          pltpu.VMEM((1, 128), I32),
          pltpu.VMEM((1, 128), I32),
          pltpu.VMEM((1, tail), I32),
          pltpu.VMEM((1, tail), I32),
          pltpu.VMEM((1, tail), I32),
          pltpu.VMEM((1, tail), I32),
          pltpu.VMEM((128, 32), F32),
          pltpu.VMEM((tail, 32), F32),
          pltpu.VMEM((128, 4), F32),
          pltpu.VMEM((128, 4), F32),
          pltpu.VMEM((tail, 4), F32),
          pltpu.VMEM((tail, 4), F32),
          pltpu.VMEM((128,), F32),
          pltpu.VMEM((128,), F32),
          pltpu.VMEM((128, 2), F32),
          pltpu.VMEM_SHARED((N, 32), F32),
          pltpu.VMEM_SHARED((N, 2), F32),
      ],
  )
  return kfn(src, dst, xlf, af, zacc, zden)


def _finalize(acc0, acc1, den0, den1, xl2, exs, bias):
  N = acc0.shape[0]
  Nb = 1000 if N % 1000 == 0 else 8
  assert N % Nb == 0

  def body(a0_ref, a1_ref, d0_ref, d1_ref, xl2_ref, exs_ref, b_ref, o_ref):
    acc = (a0_ref[...], a1_ref[...])
    den = (d0_ref[...], d1_ref[...])
    tot = None
    for h in range(4):
      p, j = h // 2, h % 2
      xlh = xl2_ref[p][:, 16 * j:16 * j + 16]
      ah = acc[p][:, 16 * j:16 * j + 16]
      eh = exs_ref[:, h:h + 1]
      num = ah + eh * xlh
      dh = den[p][:, j:j + 1] + eh
      w = num / dh
      tot = w if tot is None else tot + w
    o_ref[...] = 0.25 * tot + b_ref[...]

  return pl.pallas_call(
      body,
      grid=(N // Nb,),
      in_specs=[
          pl.BlockSpec((Nb, 32), lambda i: (i, 0)),
          pl.BlockSpec((Nb, 32), lambda i: (i, 0)),
          pl.BlockSpec((Nb, 2), lambda i: (i, 0)),
          pl.BlockSpec((Nb, 2), lambda i: (i, 0)),
          pl.BlockSpec((2, Nb, 32), lambda i: (0, i, 0)),
          pl.BlockSpec((Nb, 4), lambda i: (i, 0)),
          pl.BlockSpec((1, 16), lambda i: (0, 0)),
      ],
      out_specs=pl.BlockSpec((Nb, 16), lambda i: (i, 0)),
      out_shape=jax.ShapeDtypeStruct((N, 16), F32),
  )(acc0, acc1, den0, den1, xl2, exs, bias)


def kernel(x, edge_index, edge_attr, h_node, c_node, h_edge, c_edge,
           W_ih_n, W_hh_n, b_ih_n, b_hh_n,
           W_ih_e, W_hh_e, b_ih_e, b_hh_e,
           W_gat, att_src, att_dst, bias_gat):
  N = x.shape[0]
  HEADS, OUT = att_src.shape[1], att_src.shape[2]

  eaT = edge_attr.T
  be = (b_ih_e + b_hh_e).reshape(32, 1)
  h2T, c2T = _edge_lstm(eaT, W_ih_e, be)
  h_e2 = h2T.T
  c_e2 = c2T.T

  src = edge_index[0]
  dst = edge_index[1]

  zs = jnp.zeros((N, 8), F32)
  zc = jnp.zeros((N, 2), F32)
  ones2 = jnp.concatenate(
      [jnp.ones((128, 1), F32), jnp.zeros((128, 1), F32)], axis=1)
  sums_p, cnt_p = _p1(h_e2, dst, zs, zc, ones2)

  WnT = W_ih_n.T
  bn = (b_ih_n + b_hh_n).reshape(1, 256)
  WgT = W_gat.T
  M = jnp.repeat(jnp.eye(HEADS, dtype=F32), OUT, axis=0)
  As = att_src[0].reshape(HEADS * OUT, 1) * M
  Ad = att_dst[0].reshape(HEADS * OUT, 1) * M
  hn, cn, xl2, atab, exs = _node_proj(
      x, WnT, bn, sums_p[0:N], sums_p[N:2 * N],
      cnt_p[0:N, 0:1], cnt_p[N:2 * N, 0:1], WgT, As, Ad)

  xlf = xl2.reshape(2 * N, 32)
  af = atab.reshape(2 * N, 4)
  zacc = jnp.zeros((N, 32), F32)
  zden = jnp.zeros((N, 2), F32)
  acc, den = _p2(src, dst, xlf, af, zacc, zden)

  out = _finalize(acc[0:N], acc[N:2 * N], den[0:N], den[N:2 * N],
                  xl2, exs, bias_gat.reshape(1, 16))

  return (out, hn[None], cn[None], h_e2[None], c_e2[None])


# speedup vs baseline: 29.8971x; 29.8971x over previous
import jax
import jax.numpy as jnp
from jax import lax
from jax.experimental import pallas as pl
from jax.experimental.pallas import tpu as pltpu
from jax.experimental.pallas import tpu_sc as plsc

F32 = jnp.float32
I32 = jnp.int32


def _row_split(n):
  r = (-(-n // 16) + 7) // 8 * 8
  last = n - 15 * r
  assert last > 0 and r % 8 == 0
  return r, last


def _edge_lstm(eaT, W, be):
  E = eaT.shape[1]
  Eb = 3200 if E % 3200 == 0 else 128
  assert E % Eb == 0

  def body(ea_ref, w_ref, b_ref, h_ref, c_ref):
    g = jnp.dot(w_ref[...], ea_ref[...], preferred_element_type=F32)
    g = g + b_ref[...]
    i = jax.nn.sigmoid(g[0:8])
    gg = jnp.tanh(g[16:24])
    o = jax.nn.sigmoid(g[24:32])
    c2 = i * gg
    c_ref[...] = c2
    h_ref[...] = o * jnp.tanh(c2)

  return pl.pallas_call(
      body,
      grid=(E // Eb,),
      in_specs=[
          pl.BlockSpec((4, Eb), lambda i: (0, i)),
          pl.BlockSpec((32, 4), lambda i: (0, 0)),
          pl.BlockSpec((32, 1), lambda i: (0, 0)),
      ],
      out_specs=[
          pl.BlockSpec((8, Eb), lambda i: (0, i)),
          pl.BlockSpec((8, Eb), lambda i: (0, i)),
      ],
      out_shape=[
          jax.ShapeDtypeStruct((8, E), F32),
          jax.ShapeDtypeStruct((8, E), F32),
      ],
  )(eaT, W, be)


def _p1(h_e2, dst, zs, zc, ones1):
  E = h_e2.shape[0]
  N = zs.shape[0]
  assert E % 32 == 0
  pw = E // 32
  nbf, tail = pw // 128, pw % 128
  assert tail > 0 and pw % 8 == 0
  rA, rB = _row_split(N)
  mesh = plsc.VectorSubcoreMesh(core_axis_name="c", subcore_axis_name="s")

  def body(he, dstn, zs_h, zc_h, ones_h, sums_out, cnt_out,
           idx, idxt, rows, rowst, ones_v, stage, sums_sp, cnt_sp):
    c = lax.axis_index("c")
    s = lax.axis_index("s")
    w = s * 2 + c
    r0 = s * rA
    pltpu.sync_copy(zc_h.at[pl.ds(0, rA)], stage)

    @pl.when(s < 15)
    def _():
      pltpu.sync_copy(zs_h.at[pl.ds(r0, rA)], sums_sp.at[pl.ds(r0, rA)])
      pltpu.sync_copy(stage, cnt_sp.at[pl.ds(r0, rA)])

    @pl.when(s == 15)
    def _():
      pltpu.sync_copy(zs_h.at[pl.ds(15 * rA, rB)], sums_sp.at[pl.ds(15 * rA, rB)])
      pltpu.sync_copy(stage.at[pl.ds(0, rB)], cnt_sp.at[pl.ds(15 * rA, rB)])

    pltpu.sync_copy(ones_h, ones_v)
    plsc.subcore_barrier()

    base = w * pw

    def bb(b, cr):
      off = base + b * 128
      pltpu.sync_copy(dstn.at[pl.ds(off, 128)], idx.at[0])
      pltpu.sync_copy(he.at[pl.ds(off, 128)], rows)
      pltpu.sync_copy(rows, sums_sp.at[idx.at[0]], add=True)
      pltpu.sync_copy(ones_v, cnt_sp.at[idx.at[0]], add=True)
      return cr

    lax.fori_loop(0, nbf, bb, 0)

    offt = base + nbf * 128
    pltpu.sync_copy(dstn.at[pl.ds(offt, tail)], idxt.at[0])
    pltpu.sync_copy(he.at[pl.ds(offt, tail)], rowst)
    pltpu.sync_copy(rowst, sums_sp.at[idxt.at[0]], add=True)
    pltpu.sync_copy(ones_v.at[pl.ds(0, tail)], cnt_sp.at[idxt.at[0]], add=True)

    plsc.subcore_barrier()
    o0 = c * N + r0

    @pl.when(s < 15)
    def _():
      pltpu.sync_copy(sums_sp.at[pl.ds(r0, rA)], sums_out.at[pl.ds(o0, rA)])
      pltpu.sync_copy(cnt_sp.at[pl.ds(r0, rA)], stage)
      pltpu.sync_copy(stage, cnt_out.at[pl.ds(o0, rA)])

    @pl.when(s == 15)
    def _():
      pltpu.sync_copy(sums_sp.at[pl.ds(15 * rA, rB)],
                      sums_out.at[pl.ds(c * N + 15 * rA, rB)])
      pltpu.sync_copy(cnt_sp.at[pl.ds(15 * rA, rB)], stage.at[pl.ds(0, rB)])
      pltpu.sync_copy(stage.at[pl.ds(0, rB)],
                      cnt_out.at[pl.ds(c * N + 15 * rA, rB)])

  kfn = pl.kernel(
      body,
      out_type=[
          jax.ShapeDtypeStruct((2 * N, 8), F32),
          jax.ShapeDtypeStruct((2 * N,), F32),
      ],
      mesh=mesh,
      compiler_params=pltpu.CompilerParams(use_tc_tiling_on_sc=False),
      scratch_types=[
          pltpu.VMEM((1, 128), I32),
          pltpu.VMEM((1, tail), I32),
          pltpu.VMEM((128, 8), F32),
          pltpu.VMEM((tail, 8), F32),
          pltpu.VMEM((128,), F32),
          pltpu.VMEM((rA,), F32),
          pltpu.VMEM_SHARED((N, 8), F32),
          pltpu.VMEM_SHARED((N,), F32),
      ],
  )
  return kfn(h_e2, dst, zs, zc, ones1)


def _node_proj(x, WnT, bn, s0, s1, c0, c1, WgT, As, Ad):
  N = x.shape[0]
  Nb = 1000 if N % 1000 == 0 else 8
  assert N % Nb == 0

  def body(x_ref, wn_ref, bn_ref, s0_ref, s1_ref, c0_ref, c1_ref,
           wg_ref, as_ref, ad_ref,
           hn_ref, cn_ref, xl2_ref, as_out, ad_out, exs_ref):
    g = jnp.dot(x_ref[...], wn_ref[...], preferred_element_type=F32)
    g = g + bn_ref[...]
    i = jax.nn.sigmoid(g[:, 0:64])
    gg = jnp.tanh(g[:, 128:192])
    o = jax.nn.sigmoid(g[:, 192:256])
    c2 = i * gg
    h2 = o * jnp.tanh(c2)
    hn_ref[...] = h2
    cn_ref[...] = c2
    sums = s0_ref[...] + s1_ref[...]
    cnt = c0_ref[...] + c1_ref[...]
    ee = sums / jnp.maximum(cnt, 1.0)
    oc = jnp.concatenate([h2, ee], axis=1)
    xl = jnp.dot(oc, wg_ref[...], preferred_element_type=F32)
    a_s = jnp.dot(xl, as_ref[...], preferred_element_type=F32)
    a_d = jnp.dot(xl, ad_ref[...], preferred_element_type=F32)
    al = a_s + a_d
    al = jnp.where(al >= 0, al, 0.2 * al)
    exs_ref[...] = jnp.exp(al)
    xl2_ref[0] = xl[:, 0:32]
    xl2_ref[1] = xl[:, 32:64]
    as_out[...] = a_s
    ad_out[...] = a_d

  return pl.pallas_call(
      body,
      grid=(N // Nb,),
      in_specs=[
          pl.BlockSpec((Nb, 128), lambda i: (i, 0)),
          pl.BlockSpec((128, 256), lambda i: (0, 0)),
          pl.BlockSpec((1, 256), lambda i: (0, 0)),
          pl.BlockSpec((Nb, 8), lambda i: (i, 0)),
          pl.BlockSpec((Nb, 8), lambda i: (i, 0)),
          pl.BlockSpec((Nb, 1), lambda i: (i, 0)),
          pl.BlockSpec((Nb, 1), lambda i: (i, 0)),
          pl.BlockSpec((72, 64), lambda i: (0, 0)),
          pl.BlockSpec((64, 4), lambda i: (0, 0)),
          pl.BlockSpec((64, 4), lambda i: (0, 0)),
      ],
      out_specs=[
          pl.BlockSpec((Nb, 64), lambda i: (i, 0)),
          pl.BlockSpec((Nb, 64), lambda i: (i, 0)),
          pl.BlockSpec((2, Nb, 32), lambda i: (0, i, 0)),
          pl.BlockSpec((Nb, 4), lambda i: (i, 0)),
          pl.BlockSpec((Nb, 4), lambda i: (i, 0)),
          pl.BlockSpec((Nb, 4), lambda i: (i, 0)),
      ],
      out_shape=[
          jax.ShapeDtypeStruct((N, 64), F32),
          jax.ShapeDtypeStruct((N, 64), F32),
          jax.ShapeDtypeStruct((2, N, 32), F32),
          jax.ShapeDtypeStruct((N, 4), F32),
          jax.ShapeDtypeStruct((N, 4), F32),
          jax.ShapeDtypeStruct((N, 4), F32),
      ],
  )(x, WnT, bn, s0, s1, c0, c1, WgT, As, Ad)


def _p2(src, dst, xlf, as0, as1, ad0, ad1, zacc, zden1):
  E = src.shape[0]
  N = zacc.shape[0]
  assert E % 16 == 0
  pt = E // 16
  nbf, tail = pt // 128, pt % 128
  assert tail > 0 and tail % 16 == 0 and pt % 8 == 0
  ng_tail = tail // 16
  rA, rB = _row_split(N)
  mesh = plsc.VectorSubcoreMesh(core_axis_name="c", subcore_axis_name="s")

  def body(srcn, dstn, xl_h, as0_h, as1_h, ad0_h, ad1_h, zacc_h, zden_h,
           acc_out, den0_out, den1_out,
           si, di, sa, da, sit, dit, sat, dat,
           xlr, xlrt, a0b, a1b, b0b, b1b, a0t, a1t, b0t, b1t,
           ex0, ex1, stage, acc_sp, den0_sp, den1_sp):
    c = lax.axis_index("c")
    s = lax.axis_index("s")
    cN = c * N
    r0 = s * rA
    pltpu.sync_copy(zden_h.at[pl.ds(0, rA)], stage)

    @pl.when(s < 15)
    def _():
      pltpu.sync_copy(zacc_h.at[pl.ds(r0, rA)], acc_sp.at[pl.ds(r0, rA)])
      pltpu.sync_copy(stage, den0_sp.at[pl.ds(r0, rA)])
      pltpu.sync_copy(stage, den1_sp.at[pl.ds(r0, rA)])

    @pl.when(s == 15)
    def _():
      pltpu.sync_copy(zacc_h.at[pl.ds(15 * rA, rB)], acc_sp.at[pl.ds(15 * rA, rB)])
      pltpu.sync_copy(stage.at[pl.ds(0, rB)], den0_sp.at[pl.ds(15 * rA, rB)])
      pltpu.sync_copy(stage.at[pl.ds(0, rB)], den1_sp.at[pl.ds(15 * rA, rB)])

    plsc.subcore_barrier()

    def adjust(raw_ref, adj_ref, n):
      for g in range(n // 16):
        v = raw_ref[0, pl.ds(g * 16, 16)]
        adj_ref[0, pl.ds(g * 16, 16)] = v + cN

    def compute_ex(p0, p1, q0, q1, n_groups):
      for g in range(n_groups):
        sl = pl.ds(g * 16, 16)
        a0 = p0[sl] + q0[sl]
        a0 = jnp.where(a0 >= 0, a0, 0.2 * a0)
        ex0[sl] = jnp.exp(a0)
        a1 = p1[sl] + q1[sl]
        a1 = jnp.where(a1 >= 0, a1, 0.2 * a1)
        ex1[sl] = jnp.exp(a1)

    def scale(xlr_ref, nb):
      for g in range(nb // 16):
        e0v = ex0[pl.ds(g * 16, 16)]
        e1v = ex1[pl.ds(g * 16, 16)]
        for j in range(16):
          e = g * 16 + j
          xlr_ref[e, pl.ds(0, 16)] = xlr_ref[e, pl.ds(0, 16)] * e0v[j]
          xlr_ref[e, pl.ds(16, 16)] = xlr_ref[e, pl.ds(16, 16)] * e1v[j]

    base = s * pt

    def bb(b, cr):
      off = base + b * 128
      pltpu.sync_copy(srcn.at[pl.ds(off, 128)], si.at[0])
      pltpu.sync_copy(dstn.at[pl.ds(off, 128)], di.at[0])
      adjust(si, sa, 128)
      adjust(di, da, 128)
      pltpu.sync_copy(xl_h.at[sa.at[0]], xlr)
      pltpu.sync_copy(as0_h.at[sa.at[0]], a0b)
      pltpu.sync_copy(as1_h.at[sa.at[0]], a1b)
      pltpu.sync_copy(ad0_h.at[da.at[0]], b0b)
      pltpu.sync_copy(ad1_h.at[da.at[0]], b1b)
      compute_ex(a0b, a1b, b0b, b1b, 8)
      scale(xlr, 128)
      pltpu.sync_copy(xlr, acc_sp.at[di.at[0]], add=True)
      pltpu.sync_copy(ex0, den0_sp.at[di.at[0]], add=True)
      pltpu.sync_copy(ex1, den1_sp.at[di.at[0]], add=True)
      return cr

    lax.fori_loop(0, nbf, bb, 0)

    offt = base + nbf * 128
    pltpu.sync_copy(srcn.at[pl.ds(offt, tail)], sit.at[0])
    pltpu.sync_copy(dstn.at[pl.ds(offt, tail)], dit.at[0])
    adjust(sit, sat, tail)
    adjust(dit, dat, tail)
    pltpu.sync_copy(xl_h.at[sat.at[0]], xlrt)
    pltpu.sync_copy(as0_h.at[sat.at[0]], a0t)
    pltpu.sync_copy(as1_h.at[sat.at[0]], a1t)
    pltpu.sync_copy(ad0_h.at[dat.at[0]], b0t)
    pltpu.sync_copy(ad1_h.at[dat.at[0]], b1t)
    compute_ex(a0t, a1t, b0t, b1t, ng_tail)
    scale(xlrt, tail)
    pltpu.sync_copy(xlrt, acc_sp.at[dit.at[0]], add=True)
    pltpu.sync_copy(ex0.at[pl.ds(0, tail)], den0_sp.at[dit.at[0]], add=True)
    pltpu.sync_copy(ex1.at[pl.ds(0, tail)], den1_sp.at[dit.at[0]], add=True)

    plsc.subcore_barrier()
    o0 = cN + r0

    @pl.when(s < 15)
    def _():
      pltpu.sync_copy(acc_sp.at[pl.ds(r0, rA)], acc_out.at[pl.ds(o0, rA)])
      pltpu.sync_copy(den0_sp.at[pl.ds(r0, rA)], stage)
      pltpu.sync_copy(stage, den0_out.at[pl.ds(o0, rA)])
      pltpu.sync_copy(den1_sp.at[pl.ds(r0, rA)], stage)
      pltpu.sync_copy(stage, den1_out.at[pl.ds(o0, rA)])

    @pl.when(s == 15)
    def _():
      pltpu.sync_copy(acc_sp.at[pl.ds(15 * rA, rB)],
                      acc_out.at[pl.ds(cN + 15 * rA, rB)])
      pltpu.sync_copy(den0_sp.at[pl.ds(15 * rA, rB)], stage.at[pl.ds(0, rB)])
      pltpu.sync_copy(stage.at[pl.ds(0, rB)],
                      den0_out.at[pl.ds(cN + 15 * rA, rB)])
      pltpu.sync_copy(den1_sp.at[pl.ds(15 * rA, rB)], stage.at[pl.ds(0, rB)])
      pltpu.sync_copy(stage.at[pl.ds(0, rB)],
                      den1_out.at[pl.ds(cN + 15 * rA, rB)])

  kfn = pl.kernel(
      body,
      out_type=[
          jax.ShapeDtypeStruct((2 * N, 32), F32),
          jax.ShapeDtypeStruct((2 * N,), F32),
          jax.ShapeDtypeStruct((2 * N,), F32),
      ],
      mesh=mesh,
      compiler_params=pltpu.CompilerParams(use_tc_tiling_on_sc=False),
      scratch_types=[
          pltpu.VMEM((1, 128), I32),
          pltpu.VMEM((1, 128), I32),
          pltpu.VMEM((1, 128), I32),
          pltpu.VMEM((1, 128), I32),
          pltpu.VMEM((1, tail), I32),
          pltpu.VMEM((1, tail), I32),
          pltpu.VMEM((1, tail), I32),
          pltpu.VMEM((1, tail), I32),
          pltpu.VMEM((128, 32), F32),
          pltpu.VMEM((tail, 32), F32),
          pltpu.VMEM((128,), F32),
          pltpu.VMEM((128,), F32),
          pltpu.VMEM((128,), F32),
          pltpu.VMEM((128,), F32),
          pltpu.VMEM((tail,), F32),
          pltpu.VMEM((tail,), F32),
          pltpu.VMEM((tail,), F32),
          pltpu.VMEM((tail,), F32),
          pltpu.VMEM((128,), F32),
          pltpu.VMEM((128,), F32),
          pltpu.VMEM((rA,), F32),
          pltpu.VMEM_SHARED((N, 32), F32),
          pltpu.VMEM_SHARED((N,), F32),
          pltpu.VMEM_SHARED((N,), F32),
      ],
  )
  return kfn(src, dst, xlf, as0, as1, ad0, ad1, zacc, zden1)


def _finalize(acc0, acc1, d00, d01, d10, d11, xl2, exs, bias):
  N = acc0.shape[0]
  Nb = 1000 if N % 1000 == 0 else 8
  assert N % Nb == 0

  def body(a0_ref, a1_ref, d00_ref, d01_ref, d10_ref, d11_ref,
           xl2_ref, exs_ref, b_ref, o_ref):
    acc = (a0_ref[...], a1_ref[...])
    den = (d00_ref[...], d01_ref[...], d10_ref[...], d11_ref[...])
    tot = None
    for h in range(4):
      p, j = h // 2, h % 2
      xlh = xl2_ref[p][:, 16 * j:16 * j + 16]
      ah = acc[p][:, 16 * j:16 * j + 16]
      eh = exs_ref[:, h:h + 1]
      num = ah + eh * xlh
      dh = den[h] + eh
      w = num / dh
      tot = w if tot is None else tot + w
    o_ref[...] = 0.25 * tot + b_ref[...]

  return pl.pallas_call(
      body,
      grid=(N // Nb,),
      in_specs=[
          pl.BlockSpec((Nb, 32), lambda i: (i, 0)),
          pl.BlockSpec((Nb, 32), lambda i: (i, 0)),
          pl.BlockSpec((Nb, 1), lambda i: (i, 0)),
          pl.BlockSpec((Nb, 1), lambda i: (i, 0)),
          pl.BlockSpec((Nb, 1), lambda i: (i, 0)),
          pl.BlockSpec((Nb, 1), lambda i: (i, 0)),
          pl.BlockSpec((2, Nb, 32), lambda i: (0, i, 0)),
          pl.BlockSpec((Nb, 4), lambda i: (i, 0)),
          pl.BlockSpec((1, 16), lambda i: (0, 0)),
      ],
      out_specs=pl.BlockSpec((Nb, 16), lambda i: (i, 0)),
      out_shape=jax.ShapeDtypeStruct((N, 16), F32),
  )(acc0, acc1, d00, d01, d10, d11, xl2, exs, bias)


def kernel(x, edge_index, edge_attr, h_node, c_node, h_edge, c_edge,
           W_ih_n, W_hh_n, b_ih_n, b_hh_n,
           W_ih_e, W_hh_e, b_ih_e, b_hh_e,
           W_gat, att_src, att_dst, bias_gat):
  N = x.shape[0]
  HEADS, OUT = att_src.shape[1], att_src.shape[2]

  eaT = edge_attr.T
  be = (b_ih_e + b_hh_e).reshape(32, 1)
  h2T, c2T = _edge_lstm(eaT, W_ih_e, be)
  h_e2 = h2T.T
  c_e2 = c2T.T

  src = edge_index[0]
  dst = edge_index[1]

  zs = jnp.zeros((N, 8), F32)
  zc = jnp.zeros((N,), F32)
  ones1 = jnp.ones((128,), F32)
  sums_p, cnt_p = _p1(h_e2, dst, zs, zc, ones1)

  WnT = W_ih_n.T
  bn = (b_ih_n + b_hh_n).reshape(1, 256)
  WgT = W_gat.T
  M = jnp.repeat(jnp.eye(HEADS, dtype=F32), OUT, axis=0)
  As = att_src[0].reshape(HEADS * OUT, 1) * M
  Ad = att_dst[0].reshape(HEADS * OUT, 1) * M
  hn, cn, xl2, a_s, a_d, exs = _node_proj(
      x, WnT, bn, sums_p[0:N], sums_p[N:2 * N],
      cnt_p[0:N].reshape(N, 1), cnt_p[N:2 * N].reshape(N, 1), WgT, As, Ad)

  xlf = xl2.reshape(2 * N, 32)
  as0 = jnp.concatenate([a_s[:, 0], a_s[:, 2]])
  as1 = jnp.concatenate([a_s[:, 1], a_s[:, 3]])
  ad0 = jnp.concatenate([a_d[:, 0], a_d[:, 2]])
  ad1 = jnp.concatenate([a_d[:, 1], a_d[:, 3]])
  zacc = jnp.zeros((N, 32), F32)
  zden1 = jnp.zeros((N,), F32)
  acc, den0, den1 = _p2(src, dst, xlf, as0, as1, ad0, ad1, zacc, zden1)

  out = _finalize(acc[0:N], acc[N:2 * N],
                  den0[0:N].reshape(N, 1), den1[0:N].reshape(N, 1),
                  den0[N:2 * N].reshape(N, 1), den1[N:2 * N].reshape(N, 1),
                  xl2, exs, bias_gat.reshape(1, 16))

  return (out, hn[None], cn[None], h_e2[None], c_e2[None])


# speedup vs baseline: 48.8012x; 1.6323x over previous
import jax
import jax.numpy as jnp
from jax import lax
from jax.experimental import pallas as pl
from jax.experimental.pallas import tpu as pltpu
from jax.experimental.pallas import tpu_sc as plsc

F32 = jnp.float32
I32 = jnp.int32


def _row_split(n):
  r = (-(-n // 16) + 7) // 8 * 8
  last = n - 15 * r
  assert last > 0 and r % 8 == 0
  return r, last


def _edge_lstm(eaT, W, be):
  E = eaT.shape[1]
  Eb = 3200 if E % 3200 == 0 else 128
  assert E % Eb == 0

  def body(ea_ref, w_ref, b_ref, h_ref, c_ref):
    g = jnp.dot(w_ref[...], ea_ref[...], preferred_element_type=F32)
    g = g + b_ref[...]
    i = jax.nn.sigmoid(g[0:8])
    gg = jnp.tanh(g[16:24])
    o = jax.nn.sigmoid(g[24:32])
    c2 = i * gg
    c_ref[...] = c2
    h_ref[...] = o * jnp.tanh(c2)

  return pl.pallas_call(
      body,
      grid=(E // Eb,),
      in_specs=[
          pl.BlockSpec((4, Eb), lambda i: (0, i)),
          pl.BlockSpec((32, 4), lambda i: (0, 0)),
          pl.BlockSpec((32, 1), lambda i: (0, 0)),
      ],
      out_specs=[
          pl.BlockSpec((8, Eb), lambda i: (0, i)),
          pl.BlockSpec((8, Eb), lambda i: (0, i)),
      ],
      out_shape=[
          jax.ShapeDtypeStruct((8, E), F32),
          jax.ShapeDtypeStruct((8, E), F32),
      ],
  )(eaT, W, be)


def _p1(h_e2, dst, zs, zc, ones1):
  E = h_e2.shape[0]
  N = zs.shape[0]
  assert E % 32 == 0
  pw = E // 32
  nbf, tail = pw // 128, pw % 128
  assert tail > 0 and pw % 8 == 0
  rA, rB = _row_split(N)
  mesh = plsc.VectorSubcoreMesh(core_axis_name="c", subcore_axis_name="s")

  def body(he, dstn, zs_h, zc_h, ones_h, sums_out, cnt_out,
           idx, idxt, rows, rowst, ones_v, stage, sums_sp, cnt_sp):
    c = lax.axis_index("c")
    s = lax.axis_index("s")
    w = s * 2 + c
    r0 = s * rA
    pltpu.sync_copy(zc_h.at[pl.ds(0, rA)], stage)

    @pl.when(s < 15)
    def _():
      pltpu.sync_copy(zs_h.at[pl.ds(r0, rA)], sums_sp.at[pl.ds(r0, rA)])
      pltpu.sync_copy(stage, cnt_sp.at[pl.ds(r0, rA)])

    @pl.when(s == 15)
    def _():
      pltpu.sync_copy(zs_h.at[pl.ds(15 * rA, rB)], sums_sp.at[pl.ds(15 * rA, rB)])
      pltpu.sync_copy(stage.at[pl.ds(0, rB)], cnt_sp.at[pl.ds(15 * rA, rB)])

    pltpu.sync_copy(ones_h, ones_v)
    plsc.subcore_barrier()

    base = w * pw

    def bb(b, cr):
      off = base + b * 128
      pltpu.sync_copy(dstn.at[pl.ds(off, 128)], idx.at[0])
      pltpu.sync_copy(he.at[pl.ds(off, 128)], rows)
      pltpu.sync_copy(rows, sums_sp.at[idx.at[0]], add=True)
      pltpu.sync_copy(ones_v, cnt_sp.at[idx.at[0]], add=True)
      return cr

    lax.fori_loop(0, nbf, bb, 0)

    offt = base + nbf * 128
    pltpu.sync_copy(dstn.at[pl.ds(offt, tail)], idxt.at[0])
    pltpu.sync_copy(he.at[pl.ds(offt, tail)], rowst)
    pltpu.sync_copy(rowst, sums_sp.at[idxt.at[0]], add=True)
    pltpu.sync_copy(ones_v.at[pl.ds(0, tail)], cnt_sp.at[idxt.at[0]], add=True)

    plsc.subcore_barrier()
    o0 = c * N + r0

    @pl.when(s < 15)
    def _():
      pltpu.sync_copy(sums_sp.at[pl.ds(r0, rA)], sums_out.at[pl.ds(o0, rA)])
      pltpu.sync_copy(cnt_sp.at[pl.ds(r0, rA)], stage)
      pltpu.sync_copy(stage, cnt_out.at[pl.ds(o0, rA)])

    @pl.when(s == 15)
    def _():
      pltpu.sync_copy(sums_sp.at[pl.ds(15 * rA, rB)],
                      sums_out.at[pl.ds(c * N + 15 * rA, rB)])
      pltpu.sync_copy(cnt_sp.at[pl.ds(15 * rA, rB)], stage.at[pl.ds(0, rB)])
      pltpu.sync_copy(stage.at[pl.ds(0, rB)],
                      cnt_out.at[pl.ds(c * N + 15 * rA, rB)])

  kfn = pl.kernel(
      body,
      out_type=[
          jax.ShapeDtypeStruct((2 * N, 8), F32),
          jax.ShapeDtypeStruct((2 * N,), F32),
      ],
      mesh=mesh,
      compiler_params=pltpu.CompilerParams(use_tc_tiling_on_sc=False),
      scratch_types=[
          pltpu.VMEM((1, 128), I32),
          pltpu.VMEM((1, tail), I32),
          pltpu.VMEM((128, 8), F32),
          pltpu.VMEM((tail, 8), F32),
          pltpu.VMEM((128,), F32),
          pltpu.VMEM((rA,), F32),
          pltpu.VMEM_SHARED((N, 8), F32),
          pltpu.VMEM_SHARED((N,), F32),
      ],
  )
  return kfn(h_e2, dst, zs, zc, ones1)


def _node_proj(x, WnT, bn, s0, s1, c0, c1, WgT, As, Ad):
  N = x.shape[0]
  Nb = 1000 if N % 1000 == 0 else 8
  assert N % Nb == 0

  def body(x_ref, wn_ref, bn_ref, s0_ref, s1_ref, c0_ref, c1_ref,
           wg_ref, as_ref, ad_ref,
           hn_ref, cn_ref, xl2_ref, as_out, ad_out, exs_ref):
    g = jnp.dot(x_ref[...], wn_ref[...], preferred_element_type=F32)
    g = g + bn_ref[...]
    i = jax.nn.sigmoid(g[:, 0:64])
    gg = jnp.tanh(g[:, 128:192])
    o = jax.nn.sigmoid(g[:, 192:256])
    c2 = i * gg
    h2 = o * jnp.tanh(c2)
    hn_ref[...] = h2
    cn_ref[...] = c2
    sums = s0_ref[...] + s1_ref[...]
    cnt = c0_ref[...] + c1_ref[...]
    ee = sums / jnp.maximum(cnt, 1.0)
    oc = jnp.concatenate([h2, ee], axis=1)
    xl = jnp.dot(oc, wg_ref[...], preferred_element_type=F32)
    a_s = jnp.dot(xl, as_ref[...], preferred_element_type=F32)
    a_d = jnp.dot(xl, ad_ref[...], preferred_element_type=F32)
    al = a_s + a_d
    al = jnp.where(al >= 0, al, 0.2 * al)
    exs_ref[...] = jnp.exp(al)
    xl2_ref[0] = xl[:, 0:32]
    xl2_ref[1] = xl[:, 32:64]
    as_out[...] = a_s
    ad_out[...] = a_d

  return pl.pallas_call(
      body,
      grid=(N // Nb,),
      in_specs=[
          pl.BlockSpec((Nb, 128), lambda i: (i, 0)),
          pl.BlockSpec((128, 256), lambda i: (0, 0)),
          pl.BlockSpec((1, 256), lambda i: (0, 0)),
          pl.BlockSpec((Nb, 8), lambda i: (i, 0)),
          pl.BlockSpec((Nb, 8), lambda i: (i, 0)),
          pl.BlockSpec((Nb, 1), lambda i: (i, 0)),
          pl.BlockSpec((Nb, 1), lambda i: (i, 0)),
          pl.BlockSpec((72, 64), lambda i: (0, 0)),
          pl.BlockSpec((64, 4), lambda i: (0, 0)),
          pl.BlockSpec((64, 4), lambda i: (0, 0)),
      ],
      out_specs=[
          pl.BlockSpec((Nb, 64), lambda i: (i, 0)),
          pl.BlockSpec((Nb, 64), lambda i: (i, 0)),
          pl.BlockSpec((2, Nb, 32), lambda i: (0, i, 0)),
          pl.BlockSpec((Nb, 4), lambda i: (i, 0)),
          pl.BlockSpec((Nb, 4), lambda i: (i, 0)),
          pl.BlockSpec((Nb, 4), lambda i: (i, 0)),
      ],
      out_shape=[
          jax.ShapeDtypeStruct((N, 64), F32),
          jax.ShapeDtypeStruct((N, 64), F32),
          jax.ShapeDtypeStruct((2, N, 32), F32),
          jax.ShapeDtypeStruct((N, 4), F32),
          jax.ShapeDtypeStruct((N, 4), F32),
          jax.ShapeDtypeStruct((N, 4), F32),
      ],
  )(x, WnT, bn, s0, s1, c0, c1, WgT, As, Ad)


def _p2(src, dst, xlf, as0, as1, ad0, ad1, zacc, zden1):
  E = src.shape[0]
  N = zacc.shape[0]
  assert E % 16 == 0
  pt = E // 16
  nbf, tail = pt // 128, pt % 128
  assert nbf >= 2 and nbf % 2 == 0
  assert tail > 0 and tail % 16 == 0 and pt % 8 == 0
  ng_tail = tail // 16
  rA, rB = _row_split(N)
  mesh = plsc.VectorSubcoreMesh(core_axis_name="c", subcore_axis_name="s")

  def body(srcn, dstn, xl_h, as0_h, as1_h, ad0_h, ad1_h, zacc_h, zden_h,
           acc_out, den0_out, den1_out,
           si0, si1, di0, di1, sa0, sa1, da0, da1, dis0, dis1,
           xlr0, xlr1, a0b0, a0b1, a1b0, a1b1, b0b0, b0b1, b1b0, b1b1,
           ex0_0, ex0_1, ex1_0, ex1_1,
           sit, dit, sat, dat, xlrt, a0t, a1t, b0t, b1t,
           stage,
           smi0, smi1, smg0, smg1, sms0, sms1,
           acc_sp, den0_sp, den1_sp):
    c = lax.axis_index("c")
    s = lax.axis_index("s")
    cN = c * N
    r0 = s * rA

    si = (si0, si1)
    di = (di0, di1)
    sa = (sa0, sa1)
    da = (da0, da1)
    dis = (dis0, dis1)
    xlr = (xlr0, xlr1)
    a0b = (a0b0, a0b1)
    a1b = (a1b0, a1b1)
    b0b = (b0b0, b0b1)
    b1b = (b1b0, b1b1)
    ex0 = (ex0_0, ex0_1)
    ex1 = (ex1_0, ex1_1)
    smi = (smi0, smi1)
    smg = (smg0, smg1)
    sms = (sms0, sms1)

    pltpu.sync_copy(zden_h.at[pl.ds(0, rA)], stage)

    @pl.when(s < 15)
    def _():
      pltpu.sync_copy(zacc_h.at[pl.ds(r0, rA)], acc_sp.at[pl.ds(r0, rA)])
      pltpu.sync_copy(stage, den0_sp.at[pl.ds(r0, rA)])
      pltpu.sync_copy(stage, den1_sp.at[pl.ds(r0, rA)])

    @pl.when(s == 15)
    def _():
      pltpu.sync_copy(zacc_h.at[pl.ds(15 * rA, rB)], acc_sp.at[pl.ds(15 * rA, rB)])
      pltpu.sync_copy(stage.at[pl.ds(0, rB)], den0_sp.at[pl.ds(15 * rA, rB)])
      pltpu.sync_copy(stage.at[pl.ds(0, rB)], den1_sp.at[pl.ds(15 * rA, rB)])

    plsc.subcore_barrier()

    base = s * pt

    def adjust(p):
      for g in range(8):
        sl = pl.ds(g * 16, 16)
        v = si[p][0, sl]
        sa[p][0, sl] = v + cN
        vd = di[p][0, sl]
        da[p][0, sl] = vd + cN
        dis[p][0, sl] = vd

    def fire_idx(p, off):
      pltpu.async_copy(srcn.at[pl.ds(off, 128)], si[p].at[0], smi[p])
      pltpu.async_copy(dstn.at[pl.ds(off, 128)], di[p].at[0], smi[p])

    def wait_idx(p, off):
      pltpu.make_async_copy(srcn.at[pl.ds(off, 128)], si[p].at[0], smi[p]).wait()
      pltpu.make_async_copy(dstn.at[pl.ds(off, 128)], di[p].at[0], smi[p]).wait()

    def fire_gathers(p):
      pltpu.async_copy(xl_h.at[sa[p].at[0]], xlr[p], smg[p])
      pltpu.async_copy(as0_h.at[sa[p].at[0]], a0b[p], smg[p])
      pltpu.async_copy(as1_h.at[sa[p].at[0]], a1b[p], smg[p])
      pltpu.async_copy(ad0_h.at[da[p].at[0]], b0b[p], smg[p])
      pltpu.async_copy(ad1_h.at[da[p].at[0]], b1b[p], smg[p])

    def wait_gathers(p):
      pltpu.make_async_copy(xl_h.at[sa[p].at[0]], xlr[p], smg[p]).wait()
      pltpu.make_async_copy(as0_h.at[sa[p].at[0]], a0b[p], smg[p]).wait()
      pltpu.make_async_copy(as1_h.at[sa[p].at[0]], a1b[p], smg[p]).wait()
      pltpu.make_async_copy(ad0_h.at[da[p].at[0]], b0b[p], smg[p]).wait()
      pltpu.make_async_copy(ad1_h.at[da[p].at[0]], b1b[p], smg[p]).wait()

    def fire_scatters(p):
      pltpu.async_copy(xlr[p], acc_sp.at[dis[p].at[0]], sms[p], add=True)
      pltpu.async_copy(ex0[p], den0_sp.at[dis[p].at[0]], sms[p], add=True)
      pltpu.async_copy(ex1[p], den1_sp.at[dis[p].at[0]], sms[p], add=True)

    def wait_scatters(p):
      pltpu.make_async_copy(xlr[p], acc_sp.at[dis[p].at[0]], sms[p]).wait()
      pltpu.make_async_copy(ex0[p], den0_sp.at[dis[p].at[0]], sms[p]).wait()
      pltpu.make_async_copy(ex1[p], den1_sp.at[dis[p].at[0]], sms[p]).wait()

    def compute(p):
      for g in range(8):
        sl = pl.ds(g * 16, 16)
        a0 = a0b[p][sl] + b0b[p][sl]
        a0 = jnp.where(a0 >= 0, a0, 0.2 * a0)
        ex0[p][sl] = jnp.exp(a0)
        a1 = a1b[p][sl] + b1b[p][sl]
        a1 = jnp.where(a1 >= 0, a1, 0.2 * a1)
        ex1[p][sl] = jnp.exp(a1)
      for g in range(8):
        e0v = ex0[p][pl.ds(g * 16, 16)]
        e1v = ex1[p][pl.ds(g * 16, 16)]
        for j in range(16):
          e = g * 16 + j
          xlr[p][e, pl.ds(0, 16)] = xlr[p][e, pl.ds(0, 16)] * e0v[j]
          xlr[p][e, pl.ds(16, 16)] = xlr[p][e, pl.ds(16, 16)] * e1v[j]

    pltpu.sync_copy(srcn.at[pl.ds(base, 128)], si[0].at[0])
    pltpu.sync_copy(dstn.at[pl.ds(base, 128)], di[0].at[0])
    adjust(0)
    fire_gathers(0)
    fire_idx(1, base + 128)

    kmax = nbf // 2 - 1

    def kb(k, cr):
      b = 2 * k
      off = base + b * 128
      wait_gathers(0)
      compute(0)
      fire_scatters(0)
      wait_idx(1, off + 128)

      @pl.when(k > 0)
      def _():
        wait_scatters(1)

      adjust(1)
      fire_gathers(1)

      @pl.when(k < kmax)
      def _():
        fire_idx(0, off + 256)

      wait_gathers(1)
      compute(1)
      fire_scatters(1)

      @pl.when(k < kmax)
      def _():
        wait_idx(0, off + 256)
        wait_scatters(0)
        adjust(0)
        fire_gathers(0)
        fire_idx(1, off + 384)

      @pl.when(k == kmax)
      def _():
        wait_scatters(0)

      return cr

    lax.fori_loop(0, nbf // 2, kb, 0)
    wait_scatters(1)

    offt = base + nbf * 128
    pltpu.sync_copy(srcn.at[pl.ds(offt, tail)], sit.at[0])
    pltpu.sync_copy(dstn.at[pl.ds(offt, tail)], dit.at[0])
    for g in range(tail // 16):
      sl = pl.ds(g * 16, 16)
      sat[0, sl] = sit[0, sl] + cN
      dat[0, sl] = dit[0, sl] + cN
    pltpu.sync_copy(xl_h.at[sat.at[0]], xlrt)
    pltpu.sync_copy(as0_h.at[sat.at[0]], a0t)
    pltpu.sync_copy(as1_h.at[sat.at[0]], a1t)
    pltpu.sync_copy(ad0_h.at[dat.at[0]], b0t)
    pltpu.sync_copy(ad1_h.at[dat.at[0]], b1t)
    for g in range(ng_tail):
      sl = pl.ds(g * 16, 16)
      a0 = a0t[sl] + b0t[sl]
      a0 = jnp.where(a0 >= 0, a0, 0.2 * a0)
      ex0_0[sl] = jnp.exp(a0)
      a1 = a1t[sl] + b1t[sl]
      a1 = jnp.where(a1 >= 0, a1, 0.2 * a1)
      ex1_0[sl] = jnp.exp(a1)
    for g in range(ng_tail):
      e0v = ex0_0[pl.ds(g * 16, 16)]
      e1v = ex1_0[pl.ds(g * 16, 16)]
      for j in range(16):
        e = g * 16 + j
        xlrt[e, pl.ds(0, 16)] = xlrt[e, pl.ds(0, 16)] * e0v[j]
        xlrt[e, pl.ds(16, 16)] = xlrt[e, pl.ds(16, 16)] * e1v[j]
    pltpu.sync_copy(xlrt, acc_sp.at[dit.at[0]], add=True)
    pltpu.sync_copy(ex0_0.at[pl.ds(0, tail)], den0_sp.at[dit.at[0]], add=True)
    pltpu.sync_copy(ex1_0.at[pl.ds(0, tail)], den1_sp.at[dit.at[0]], add=True)

    plsc.subcore_barrier()
    o0 = cN + r0

    @pl.when(s < 15)
    def _():
      pltpu.sync_copy(acc_sp.at[pl.ds(r0, rA)], acc_out.at[pl.ds(o0, rA)])
      pltpu.sync_copy(den0_sp.at[pl.ds(r0, rA)], stage)
      pltpu.sync_copy(stage, den0_out.at[pl.ds(o0, rA)])
      pltpu.sync_copy(den1_sp.at[pl.ds(r0, rA)], stage)
      pltpu.sync_copy(stage, den1_out.at[pl.ds(o0, rA)])

    @pl.when(s == 15)
    def _():
      pltpu.sync_copy(acc_sp.at[pl.ds(15 * rA, rB)],
                      acc_out.at[pl.ds(cN + 15 * rA, rB)])
      pltpu.sync_copy(den0_sp.at[pl.ds(15 * rA, rB)], stage.at[pl.ds(0, rB)])
      pltpu.sync_copy(stage.at[pl.ds(0, rB)],
                      den0_out.at[pl.ds(cN + 15 * rA, rB)])
      pltpu.sync_copy(den1_sp.at[pl.ds(15 * rA, rB)], stage.at[pl.ds(0, rB)])
      pltpu.sync_copy(stage.at[pl.ds(0, rB)],
                      den1_out.at[pl.ds(cN + 15 * rA, rB)])

  kfn = pl.kernel(
      body,
      out_type=[
          jax.ShapeDtypeStruct((2 * N, 32), F32),
          jax.ShapeDtypeStruct((2 * N,), F32),
          jax.ShapeDtypeStruct((2 * N,), F32),
      ],
      mesh=mesh,
      compiler_params=pltpu.CompilerParams(use_tc_tiling_on_sc=False),
      scratch_types=(
          [pltpu.VMEM((1, 128), I32) for _ in range(10)]
          + [pltpu.VMEM((128, 32), F32), pltpu.VMEM((128, 32), F32)]
          + [pltpu.VMEM((128,), F32) for _ in range(8)]
          + [pltpu.VMEM((128,), F32) for _ in range(4)]
          + [pltpu.VMEM((1, tail), I32) for _ in range(4)]
          + [pltpu.VMEM((tail, 32), F32)]
          + [pltpu.VMEM((tail,), F32) for _ in range(4)]
          + [pltpu.VMEM((rA,), F32)]
          + [pltpu.SemaphoreType.DMA for _ in range(6)]
          + [
              pltpu.VMEM_SHARED((N, 32), F32),
              pltpu.VMEM_SHARED((N,), F32),
              pltpu.VMEM_SHARED((N,), F32),
          ]
      ),
  )
  return kfn(src, dst, xlf, as0, as1, ad0, ad1, zacc, zden1)


def _finalize(acc0, acc1, d00, d01, d10, d11, xl2, exs, bias):
  N = acc0.shape[0]
  Nb = 1000 if N % 1000 == 0 else 8
  assert N % Nb == 0

  def body(a0_ref, a1_ref, d00_ref, d01_ref, d10_ref, d11_ref,
           xl2_ref, exs_ref, b_ref, o_ref):
    acc = (a0_ref[...], a1_ref[...])
    den = (d00_ref[...], d01_ref[...], d10_ref[...], d11_ref[...])
    tot = None
    for h in range(4):
      p, j = h // 2, h % 2
      xlh = xl2_ref[p][:, 16 * j:16 * j + 16]
      ah = acc[p][:, 16 * j:16 * j + 16]
      eh = exs_ref[:, h:h + 1]
      num = ah + eh * xlh
      dh = den[h] + eh
      w = num / dh
      tot = w if tot is None else tot + w
    o_ref[...] = 0.25 * tot + b_ref[...]

  return pl.pallas_call(
      body,
      grid=(N // Nb,),
      in_specs=[
          pl.BlockSpec((Nb, 32), lambda i: (i, 0)),
          pl.BlockSpec((Nb, 32), lambda i: (i, 0)),
          pl.BlockSpec((Nb, 1), lambda i: (i, 0)),
          pl.BlockSpec((Nb, 1), lambda i: (i, 0)),
          pl.BlockSpec((Nb, 1), lambda i: (i, 0)),
          pl.BlockSpec((Nb, 1), lambda i: (i, 0)),
          pl.BlockSpec((2, Nb, 32), lambda i: (0, i, 0)),
          pl.BlockSpec((Nb, 4), lambda i: (i, 0)),
          pl.BlockSpec((1, 16), lambda i: (0, 0)),
      ],
      out_specs=pl.BlockSpec((Nb, 16), lambda i: (i, 0)),
      out_shape=jax.ShapeDtypeStruct((N, 16), F32),
  )(acc0, acc1, d00, d01, d10, d11, xl2, exs, bias)


def kernel(x, edge_index, edge_attr, h_node, c_node, h_edge, c_edge,
           W_ih_n, W_hh_n, b_ih_n, b_hh_n,
           W_ih_e, W_hh_e, b_ih_e, b_hh_e,
           W_gat, att_src, att_dst, bias_gat):
  N = x.shape[0]
  HEADS, OUT = att_src.shape[1], att_src.shape[2]

  eaT = edge_attr.T
  be = (b_ih_e + b_hh_e).reshape(32, 1)
  h2T, c2T = _edge_lstm(eaT, W_ih_e, be)
  h_e2 = h2T.T
  c_e2 = c2T.T

  src = edge_index[0]
  dst = edge_index[1]

  zs = jnp.zeros((N, 8), F32)
  zc = jnp.zeros((N,), F32)
  ones1 = jnp.ones((128,), F32)
  sums_p, cnt_p = _p1(h_e2, dst, zs, zc, ones1)

  WnT = W_ih_n.T
  bn = (b_ih_n + b_hh_n).reshape(1, 256)
  WgT = W_gat.T
  M = jnp.repeat(jnp.eye(HEADS, dtype=F32), OUT, axis=0)
  As = att_src[0].reshape(HEADS * OUT, 1) * M
  Ad = att_dst[0].reshape(HEADS * OUT, 1) * M
  hn, cn, xl2, a_s, a_d, exs = _node_proj(
      x, WnT, bn, sums_p[0:N], sums_p[N:2 * N],
      cnt_p[0:N].reshape(N, 1), cnt_p[N:2 * N].reshape(N, 1), WgT, As, Ad)

  xlf = xl2.reshape(2 * N, 32)
  as0 = jnp.concatenate([a_s[:, 0], a_s[:, 2]])
  as1 = jnp.concatenate([a_s[:, 1], a_s[:, 3]])
  ad0 = jnp.concatenate([a_d[:, 0], a_d[:, 2]])
  ad1 = jnp.concatenate([a_d[:, 1], a_d[:, 3]])
  zacc = jnp.zeros((N, 32), F32)
  zden1 = jnp.zeros((N,), F32)
  acc, den0, den1 = _p2(src, dst, xlf, as0, as1, ad0, ad1, zacc, zden1)

  out = _finalize(acc[0:N], acc[N:2 * N],
                  den0[0:N].reshape(N, 1), den1[0:N].reshape(N, 1),
                  den0[N:2 * N].reshape(N, 1), den1[N:2 * N].reshape(N, 1),
                  xl2, exs, bias_gat.reshape(1, 16))

  return (out, hn[None], cn[None], h_e2[None], c_e2[None])


# speedup vs baseline: 51.7023x; 1.0594x over previous
import jax
import jax.numpy as jnp
from jax import lax
from jax.experimental import pallas as pl
from jax.experimental.pallas import tpu as pltpu
from jax.experimental.pallas import tpu_sc as plsc

F32 = jnp.float32
I32 = jnp.int32


def _row_split(n):
  r = (-(-n // 16) + 7) // 8 * 8
  last = n - 15 * r
  assert last > 0 and r % 8 == 0
  return r, last


def _edge_lstm(eaT, W, be):
  E = eaT.shape[1]
  Eb = 3200 if E % 3200 == 0 else 128
  assert E % Eb == 0

  def body(ea_ref, w_ref, b_ref, h_ref, c_ref):
    g = jnp.dot(w_ref[...], ea_ref[...], preferred_element_type=F32)
    g = g + b_ref[...]
    i = jax.nn.sigmoid(g[0:8])
    gg = jnp.tanh(g[16:24])
    o = jax.nn.sigmoid(g[24:32])
    c2 = i * gg
    c_ref[...] = c2
    h_ref[...] = o * jnp.tanh(c2)

  return pl.pallas_call(
      body,
      grid=(E // Eb,),
      in_specs=[
          pl.BlockSpec((4, Eb), lambda i: (0, i)),
          pl.BlockSpec((32, 4), lambda i: (0, 0)),
          pl.BlockSpec((32, 1), lambda i: (0, 0)),
      ],
      out_specs=[
          pl.BlockSpec((8, Eb), lambda i: (0, i)),
          pl.BlockSpec((8, Eb), lambda i: (0, i)),
      ],
      out_shape=[
          jax.ShapeDtypeStruct((8, E), F32),
          jax.ShapeDtypeStruct((8, E), F32),
      ],
  )(eaT, W, be)


def _p1(h_e2, dst, zs, zc, ones1):
  E = h_e2.shape[0]
  N = zs.shape[0]
  assert E % 32 == 0
  pw = E // 32
  nbf, tail = pw // 128, pw % 128
  assert nbf >= 2 and tail > 0 and pw % 8 == 0
  rA, rB = _row_split(N)
  mesh = plsc.VectorSubcoreMesh(core_axis_name="c", subcore_axis_name="s")

  def body(he, dstn, zs_h, zc_h, ones_h, sums_out, cnt_out,
           idx0, idx1, rows0, rows1, idxt, rowst, ones_v, stage,
           sl0, sl1, sc0, sc1, sums_sp, cnt_sp):
    c = lax.axis_index("c")
    s = lax.axis_index("s")
    w = s * 2 + c
    r0 = s * rA
    idx = (idx0, idx1)
    rows = (rows0, rows1)
    sml = (sl0, sl1)
    smc = (sc0, sc1)
    pltpu.sync_copy(zc_h.at[pl.ds(0, rA)], stage)

    @pl.when(s < 15)
    def _():
      pltpu.sync_copy(zs_h.at[pl.ds(r0, rA)], sums_sp.at[pl.ds(r0, rA)])
      pltpu.sync_copy(stage, cnt_sp.at[pl.ds(r0, rA)])

    @pl.when(s == 15)
    def _():
      pltpu.sync_copy(zs_h.at[pl.ds(15 * rA, rB)], sums_sp.at[pl.ds(15 * rA, rB)])
      pltpu.sync_copy(stage.at[pl.ds(0, rB)], cnt_sp.at[pl.ds(15 * rA, rB)])

    pltpu.sync_copy(ones_h, ones_v)
    plsc.subcore_barrier()

    base = w * pw

    def fire_loads(p, off):
      pltpu.async_copy(dstn.at[pl.ds(off, 128)], idx[p].at[0], sml[p])
      pltpu.async_copy(he.at[pl.ds(off, 128)], rows[p], sml[p])

    def wait_loads(p, off):
      pltpu.make_async_copy(dstn.at[pl.ds(off, 128)], idx[p].at[0], sml[p]).wait()
      pltpu.make_async_copy(he.at[pl.ds(off, 128)], rows[p], sml[p]).wait()

    def fire_scatters(p):
      pltpu.async_copy(rows[p], sums_sp.at[idx[p].at[0]], smc[p], add=True)
      pltpu.async_copy(ones_v, cnt_sp.at[idx[p].at[0]], smc[p], add=True)

    def wait_scatters(p):
      pltpu.make_async_copy(rows[p], sums_sp.at[idx[p].at[0]], smc[p]).wait()
      pltpu.make_async_copy(ones_v, cnt_sp.at[idx[p].at[0]], smc[p]).wait()

    fire_loads(0, base)

    def kb(k, cr):
      b0 = 2 * k
      off0 = base + b0 * 128
      wait_loads(0, off0)

      @pl.when(k > 0)
      def _():
        wait_scatters(1)

      fire_loads(1, off0 + 128)
      fire_scatters(0)

      wait_loads(1, off0 + 128)
      wait_scatters(0)

      @pl.when(k + 1 < (nbf + 1) // 2)
      def _():
        fire_loads(0, off0 + 256)

      fire_scatters(1)
      return cr

    nk = nbf // 2
    lax.fori_loop(0, nk, kb, 0)
    rem = nbf - 2 * nk
    if rem:
      offr = base + 2 * nk * 128
      wait_loads(0, offr)
      wait_scatters(1)
      fire_scatters(0)
      wait_scatters(0)
    else:
      wait_scatters(1)

    offt = base + nbf * 128
    pltpu.sync_copy(dstn.at[pl.ds(offt, tail)], idxt.at[0])
    pltpu.sync_copy(he.at[pl.ds(offt, tail)], rowst)
    pltpu.sync_copy(rowst, sums_sp.at[idxt.at[0]], add=True)
    pltpu.sync_copy(ones_v.at[pl.ds(0, tail)], cnt_sp.at[idxt.at[0]], add=True)

    plsc.subcore_barrier()
    o0 = c * N + r0

    @pl.when(s < 15)
    def _():
      pltpu.sync_copy(sums_sp.at[pl.ds(r0, rA)], sums_out.at[pl.ds(o0, rA)])
      pltpu.sync_copy(cnt_sp.at[pl.ds(r0, rA)], stage)
      pltpu.sync_copy(stage, cnt_out.at[pl.ds(o0, rA)])

    @pl.when(s == 15)
    def _():
      pltpu.sync_copy(sums_sp.at[pl.ds(15 * rA, rB)],
                      sums_out.at[pl.ds(c * N + 15 * rA, rB)])
      pltpu.sync_copy(cnt_sp.at[pl.ds(15 * rA, rB)], stage.at[pl.ds(0, rB)])
      pltpu.sync_copy(stage.at[pl.ds(0, rB)],
                      cnt_out.at[pl.ds(c * N + 15 * rA, rB)])

  kfn = pl.kernel(
      body,
      out_type=[
          jax.ShapeDtypeStruct((2 * N, 8), F32),
          jax.ShapeDtypeStruct((2 * N,), F32),
      ],
      mesh=mesh,
      compiler_params=pltpu.CompilerParams(use_tc_tiling_on_sc=False),
      scratch_types=[
          pltpu.VMEM((1, 128), I32),
          pltpu.VMEM((1, 128), I32),
          pltpu.VMEM((128, 8), F32),
          pltpu.VMEM((128, 8), F32),
          pltpu.VMEM((1, tail), I32),
          pltpu.VMEM((tail, 8), F32),
          pltpu.VMEM((128,), F32),
          pltpu.VMEM((rA,), F32),
          pltpu.SemaphoreType.DMA,
          pltpu.SemaphoreType.DMA,
          pltpu.SemaphoreType.DMA,
          pltpu.SemaphoreType.DMA,
          pltpu.VMEM_SHARED((N, 8), F32),
          pltpu.VMEM_SHARED((N,), F32),
      ],
  )
  return kfn(h_e2, dst, zs, zc, ones1)


def _node_proj(x, WnT, bn, s0, s1, c0, c1, WgT, As, Ad):
  N = x.shape[0]
  Nb = 1000 if N % 1000 == 0 else 8
  assert N % Nb == 0

  def body(x_ref, wn_ref, bn_ref, s0_ref, s1_ref, c0_ref, c1_ref,
           wg_ref, as_ref, ad_ref,
           hn_ref, cn_ref, xl2_ref, as_out, ad_out, exs_ref):
    g = jnp.dot(x_ref[...], wn_ref[...], preferred_element_type=F32)
    g = g + bn_ref[...]
    i = jax.nn.sigmoid(g[:, 0:64])
    gg = jnp.tanh(g[:, 128:192])
    o = jax.nn.sigmoid(g[:, 192:256])
    c2 = i * gg
    h2 = o * jnp.tanh(c2)
    hn_ref[...] = h2
    cn_ref[...] = c2
    sums = s0_ref[...] + s1_ref[...]
    cnt = c0_ref[...] + c1_ref[...]
    ee = sums / jnp.maximum(cnt, 1.0)
    oc = jnp.concatenate([h2, ee], axis=1)
    xl = jnp.dot(oc, wg_ref[...], preferred_element_type=F32)
    a_s = jnp.dot(xl, as_ref[...], preferred_element_type=F32)
    a_d = jnp.dot(xl, ad_ref[...], preferred_element_type=F32)
    al = a_s + a_d
    al = jnp.where(al >= 0, al, 0.2 * al)
    exs_ref[...] = jnp.exp(al)
    xl2_ref[0] = xl[:, 0:32]
    xl2_ref[1] = xl[:, 32:64]
    as_out[...] = a_s
    ad_out[...] = a_d

  return pl.pallas_call(
      body,
      grid=(N // Nb,),
      in_specs=[
          pl.BlockSpec((Nb, 128), lambda i: (i, 0)),
          pl.BlockSpec((128, 256), lambda i: (0, 0)),
          pl.BlockSpec((1, 256), lambda i: (0, 0)),
          pl.BlockSpec((Nb, 8), lambda i: (i, 0)),
          pl.BlockSpec((Nb, 8), lambda i: (i, 0)),
          pl.BlockSpec((Nb, 1), lambda i: (i, 0)),
          pl.BlockSpec((Nb, 1), lambda i: (i, 0)),
          pl.BlockSpec((72, 64), lambda i: (0, 0)),
          pl.BlockSpec((64, 4), lambda i: (0, 0)),
          pl.BlockSpec((64, 4), lambda i: (0, 0)),
      ],
      out_specs=[
          pl.BlockSpec((Nb, 64), lambda i: (i, 0)),
          pl.BlockSpec((Nb, 64), lambda i: (i, 0)),
          pl.BlockSpec((2, Nb, 32), lambda i: (0, i, 0)),
          pl.BlockSpec((Nb, 4), lambda i: (i, 0)),
          pl.BlockSpec((Nb, 4), lambda i: (i, 0)),
          pl.BlockSpec((Nb, 4), lambda i: (i, 0)),
      ],
      out_shape=[
          jax.ShapeDtypeStruct((N, 64), F32),
          jax.ShapeDtypeStruct((N, 64), F32),
          jax.ShapeDtypeStruct((2, N, 32), F32),
          jax.ShapeDtypeStruct((N, 4), F32),
          jax.ShapeDtypeStruct((N, 4), F32),
          jax.ShapeDtypeStruct((N, 4), F32),
      ],
  )(x, WnT, bn, s0, s1, c0, c1, WgT, As, Ad)


def _p2(src, dst, xlf, as0, as1, ad0, ad1, zacc, zden1):
  E = src.shape[0]
  N = zacc.shape[0]
  assert E % 16 == 0
  pt = E // 16
  nbf, tail = pt // 128, pt % 128
  assert nbf >= 2 and nbf % 2 == 0
  assert tail > 0 and tail % 16 == 0 and pt % 8 == 0
  ng_tail = tail // 16
  rA, rB = _row_split(N)
  mesh = plsc.VectorSubcoreMesh(core_axis_name="c", subcore_axis_name="s")

  def body(srcn, dstn, xl_h, as0_h, as1_h, ad0_h, ad1_h, zacc_h, zden_h,
           acc_out, den0_out, den1_out,
           si0, si1, di0, di1, sa0, sa1, da0, da1, dis0, dis1,
           xlr0, xlr1, a0b0, a0b1, a1b0, a1b1, b0b0, b0b1, b1b0, b1b1,
           ex0_0, ex0_1, ex1_0, ex1_1,
           sit, dit, sat, dat, xlrt, a0t, a1t, b0t, b1t,
           stage,
           smi0, smi1, smg0, smg1, sms0, sms1,
           acc_sp, den0_sp, den1_sp):
    c = lax.axis_index("c")
    s = lax.axis_index("s")
    cN = c * N
    r0 = s * rA

    si = (si0, si1)
    di = (di0, di1)
    sa = (sa0, sa1)
    da = (da0, da1)
    dis = (dis0, dis1)
    xlr = (xlr0, xlr1)
    a0b = (a0b0, a0b1)
    a1b = (a1b0, a1b1)
    b0b = (b0b0, b0b1)
    b1b = (b1b0, b1b1)
    ex0 = (ex0_0, ex0_1)
    ex1 = (ex1_0, ex1_1)
    smi = (smi0, smi1)
    smg = (smg0, smg1)
    sms = (sms0, sms1)

    pltpu.sync_copy(zden_h.at[pl.ds(0, rA)], stage)

    @pl.when(s < 15)
    def _():
      pltpu.sync_copy(zacc_h.at[pl.ds(r0, rA)], acc_sp.at[pl.ds(r0, rA)])
      pltpu.sync_copy(stage, den0_sp.at[pl.ds(r0, rA)])
      pltpu.sync_copy(stage, den1_sp.at[pl.ds(r0, rA)])

    @pl.when(s == 15)
    def _():
      pltpu.sync_copy(zacc_h.at[pl.ds(15 * rA, rB)], acc_sp.at[pl.ds(15 * rA, rB)])
      pltpu.sync_copy(stage.at[pl.ds(0, rB)], den0_sp.at[pl.ds(15 * rA, rB)])
      pltpu.sync_copy(stage.at[pl.ds(0, rB)], den1_sp.at[pl.ds(15 * rA, rB)])

    plsc.subcore_barrier()

    base = s * pt

    def adjust(p):
      for g in range(8):
        sl = pl.ds(g * 16, 16)
        v = si[p][0, sl]
        sa[p][0, sl] = v + cN
        vd = di[p][0, sl]
        da[p][0, sl] = vd + cN
        dis[p][0, sl] = vd

    def fire_idx(p, off):
      pltpu.async_copy(srcn.at[pl.ds(off, 128)], si[p].at[0], smi[p])
      pltpu.async_copy(dstn.at[pl.ds(off, 128)], di[p].at[0], smi[p])

    def wait_idx(p, off):
      pltpu.make_async_copy(srcn.at[pl.ds(off, 128)], si[p].at[0], smi[p]).wait()
      pltpu.make_async_copy(dstn.at[pl.ds(off, 128)], di[p].at[0], smi[p]).wait()

    def fire_gathers(p):
      pltpu.async_copy(xl_h.at[sa[p].at[0]], xlr[p], smg[p])
      pltpu.async_copy(as0_h.at[sa[p].at[0]], a0b[p], smg[p])
      pltpu.async_copy(as1_h.at[sa[p].at[0]], a1b[p], smg[p])
      pltpu.async_copy(ad0_h.at[da[p].at[0]], b0b[p], smg[p])
      pltpu.async_copy(ad1_h.at[da[p].at[0]], b1b[p], smg[p])

    def wait_gathers(p):
      pltpu.make_async_copy(xl_h.at[sa[p].at[0]], xlr[p], smg[p]).wait()
      pltpu.make_async_copy(as0_h.at[sa[p].at[0]], a0b[p], smg[p]).wait()
      pltpu.make_async_copy(as1_h.at[sa[p].at[0]], a1b[p], smg[p]).wait()
      pltpu.make_async_copy(ad0_h.at[da[p].at[0]], b0b[p], smg[p]).wait()
      pltpu.make_async_copy(ad1_h.at[da[p].at[0]], b1b[p], smg[p]).wait()

    def fire_scatters(p):
      pltpu.async_copy(xlr[p], acc_sp.at[dis[p].at[0]], sms[p], add=True)
      pltpu.async_copy(ex0[p], den0_sp.at[dis[p].at[0]], sms[p], add=True)
      pltpu.async_copy(ex1[p], den1_sp.at[dis[p].at[0]], sms[p], add=True)

    def wait_scatters(p):
      pltpu.make_async_copy(xlr[p], acc_sp.at[dis[p].at[0]], sms[p]).wait()
      pltpu.make_async_copy(ex0[p], den0_sp.at[dis[p].at[0]], sms[p]).wait()
      pltpu.make_async_copy(ex1[p], den1_sp.at[dis[p].at[0]], sms[p]).wait()

    def compute(p):
      for g in range(8):
        sl = pl.ds(g * 16, 16)
        a0 = a0b[p][sl] + b0b[p][sl]
        a0 = jnp.where(a0 >= 0, a0, 0.2 * a0)
        ex0[p][sl] = jnp.exp(a0)
        a1 = a1b[p][sl] + b1b[p][sl]
        a1 = jnp.where(a1 >= 0, a1, 0.2 * a1)
        ex1[p][sl] = jnp.exp(a1)
      for g in range(8):
        e0v = ex0[p][pl.ds(g * 16, 16)]
        e1v = ex1[p][pl.ds(g * 16, 16)]
        for j in range(16):
          e = g * 16 + j
          xlr[p][e, pl.ds(0, 16)] = xlr[p][e, pl.ds(0, 16)] * e0v[j]
          xlr[p][e, pl.ds(16, 16)] = xlr[p][e, pl.ds(16, 16)] * e1v[j]

    pltpu.sync_copy(srcn.at[pl.ds(base, 128)], si[0].at[0])
    pltpu.sync_copy(dstn.at[pl.ds(base, 128)], di[0].at[0])
    adjust(0)
    fire_gathers(0)
    fire_idx(1, base + 128)

    kmax = nbf // 2 - 1

    def kb(k, cr):
      b = 2 * k
      off = base + b * 128
      wait_gathers(0)
      compute(0)
      fire_scatters(0)
      wait_idx(1, off + 128)

      @pl.when(k > 0)
      def _():
        wait_scatters(1)

      adjust(1)
      fire_gathers(1)

      @pl.when(k < kmax)
      def _():
        fire_idx(0, off + 256)

      wait_gathers(1)
      compute(1)
      fire_scatters(1)

      @pl.when(k < kmax)
      def _():
        wait_idx(0, off + 256)
        wait_scatters(0)
        adjust(0)
        fire_gathers(0)
        fire_idx(1, off + 384)

      @pl.when(k == kmax)
      def _():
        wait_scatters(0)

      return cr

    lax.fori_loop(0, nbf // 2, kb, 0)
    wait_scatters(1)

    offt = base + nbf * 128
    pltpu.sync_copy(srcn.at[pl.ds(offt, tail)], sit.at[0])
    pltpu.sync_copy(dstn.at[pl.ds(offt, tail)], dit.at[0])
    for g in range(tail // 16):
      sl = pl.ds(g * 16, 16)
      sat[0, sl] = sit[0, sl] + cN
      dat[0, sl] = dit[0, sl] + cN
    pltpu.sync_copy(xl_h.at[sat.at[0]], xlrt)
    pltpu.sync_copy(as0_h.at[sat.at[0]], a0t)
    pltpu.sync_copy(as1_h.at[sat.at[0]], a1t)
    pltpu.sync_copy(ad0_h.at[dat.at[0]], b0t)
    pltpu.sync_copy(ad1_h.at[dat.at[0]], b1t)
    for g in range(ng_tail):
      sl = pl.ds(g * 16, 16)
      a0 = a0t[sl] + b0t[sl]
      a0 = jnp.where(a0 >= 0, a0, 0.2 * a0)
      ex0_0[sl] = jnp.exp(a0)
      a1 = a1t[sl] + b1t[sl]
      a1 = jnp.where(a1 >= 0, a1, 0.2 * a1)
      ex1_0[sl] = jnp.exp(a1)
    for g in range(ng_tail):
      e0v = ex0_0[pl.ds(g * 16, 16)]
      e1v = ex1_0[pl.ds(g * 16, 16)]
      for j in range(16):
        e = g * 16 + j
        xlrt[e, pl.ds(0, 16)] = xlrt[e, pl.ds(0, 16)] * e0v[j]
        xlrt[e, pl.ds(16, 16)] = xlrt[e, pl.ds(16, 16)] * e1v[j]
    pltpu.sync_copy(xlrt, acc_sp.at[dit.at[0]], add=True)
    pltpu.sync_copy(ex0_0.at[pl.ds(0, tail)], den0_sp.at[dit.at[0]], add=True)
    pltpu.sync_copy(ex1_0.at[pl.ds(0, tail)], den1_sp.at[dit.at[0]], add=True)

    plsc.subcore_barrier()
    o0 = cN + r0

    @pl.when(s < 15)
    def _():
      pltpu.sync_copy(acc_sp.at[pl.ds(r0, rA)], acc_out.at[pl.ds(o0, rA)])
      pltpu.sync_copy(den0_sp.at[pl.ds(r0, rA)], stage)
      pltpu.sync_copy(stage, den0_out.at[pl.ds(o0, rA)])
      pltpu.sync_copy(den1_sp.at[pl.ds(r0, rA)], stage)
      pltpu.sync_copy(stage, den1_out.at[pl.ds(o0, rA)])

    @pl.when(s == 15)
    def _():
      pltpu.sync_copy(acc_sp.at[pl.ds(15 * rA, rB)],
                      acc_out.at[pl.ds(cN + 15 * rA, rB)])
      pltpu.sync_copy(den0_sp.at[pl.ds(15 * rA, rB)], stage.at[pl.ds(0, rB)])
      pltpu.sync_copy(stage.at[pl.ds(0, rB)],
                      den0_out.at[pl.ds(cN + 15 * rA, rB)])
      pltpu.sync_copy(den1_sp.at[pl.ds(15 * rA, rB)], stage.at[pl.ds(0, rB)])
      pltpu.sync_copy(stage.at[pl.ds(0, rB)],
                      den1_out.at[pl.ds(cN + 15 * rA, rB)])

  kfn = pl.kernel(
      body,
      out_type=[
          jax.ShapeDtypeStruct((2 * N, 32), F32),
          jax.ShapeDtypeStruct((2 * N,), F32),
          jax.ShapeDtypeStruct((2 * N,), F32),
      ],
      mesh=mesh,
      compiler_params=pltpu.CompilerParams(use_tc_tiling_on_sc=False),
      scratch_types=(
          [pltpu.VMEM((1, 128), I32) for _ in range(10)]
          + [pltpu.VMEM((128, 32), F32), pltpu.VMEM((128, 32), F32)]
          + [pltpu.VMEM((128,), F32) for _ in range(8)]
          + [pltpu.VMEM((128,), F32) for _ in range(4)]
          + [pltpu.VMEM((1, tail), I32) for _ in range(4)]
          + [pltpu.VMEM((tail, 32), F32)]
          + [pltpu.VMEM((tail,), F32) for _ in range(4)]
          + [pltpu.VMEM((rA,), F32)]
          + [pltpu.SemaphoreType.DMA for _ in range(6)]
          + [
              pltpu.VMEM_SHARED((N, 32), F32),
              pltpu.VMEM_SHARED((N,), F32),
              pltpu.VMEM_SHARED((N,), F32),
          ]
      ),
  )
  return kfn(src, dst, xlf, as0, as1, ad0, ad1, zacc, zden1)


def _finalize(acc0, acc1, d00, d01, d10, d11, xl2, exs, bias):
  N = acc0.shape[0]
  Nb = 1000 if N % 1000 == 0 else 8
  assert N % Nb == 0

  def body(a0_ref, a1_ref, d00_ref, d01_ref, d10_ref, d11_ref,
           xl2_ref, exs_ref, b_ref, o_ref):
    acc = (a0_ref[...], a1_ref[...])
    den = (d00_ref[...], d01_ref[...], d10_ref[...], d11_ref[...])
    tot = None
    for h in range(4):
      p, j = h // 2, h % 2
      xlh = xl2_ref[p][:, 16 * j:16 * j + 16]
      ah = acc[p][:, 16 * j:16 * j + 16]
      eh = exs_ref[:, h:h + 1]
      num = ah + eh * xlh
      dh = den[h] + eh
      w = num / dh
      tot = w if tot is None else tot + w
    o_ref[...] = 0.25 * tot + b_ref[...]

  return pl.pallas_call(
      body,
      grid=(N // Nb,),
      in_specs=[
          pl.BlockSpec((Nb, 32), lambda i: (i, 0)),
          pl.BlockSpec((Nb, 32), lambda i: (i, 0)),
          pl.BlockSpec((Nb, 1), lambda i: (i, 0)),
          pl.BlockSpec((Nb, 1), lambda i: (i, 0)),
          pl.BlockSpec((Nb, 1), lambda i: (i, 0)),
          pl.BlockSpec((Nb, 1), lambda i: (i, 0)),
          pl.BlockSpec((2, Nb, 32), lambda i: (0, i, 0)),
          pl.BlockSpec((Nb, 4), lambda i: (i, 0)),
          pl.BlockSpec((1, 16), lambda i: (0, 0)),
      ],
      out_specs=pl.BlockSpec((Nb, 16), lambda i: (i, 0)),
      out_shape=jax.ShapeDtypeStruct((N, 16), F32),
  )(acc0, acc1, d00, d01, d10, d11, xl2, exs, bias)


def kernel(x, edge_index, edge_attr, h_node, c_node, h_edge, c_edge,
           W_ih_n, W_hh_n, b_ih_n, b_hh_n,
           W_ih_e, W_hh_e, b_ih_e, b_hh_e,
           W_gat, att_src, att_dst, bias_gat):
  N = x.shape[0]
  HEADS, OUT = att_src.shape[1], att_src.shape[2]

  eaT = edge_attr.T
  be = (b_ih_e + b_hh_e).reshape(32, 1)
  h2T, c2T = _edge_lstm(eaT, W_ih_e, be)
  h_e2 = h2T.T
  c_e2 = c2T.T

  src = edge_index[0]
  dst = edge_index[1]

  zs = jnp.zeros((N, 8), F32)
  zc = jnp.zeros((N,), F32)
  ones1 = jnp.ones((128,), F32)
  sums_p, cnt_p = _p1(h_e2, dst, zs, zc, ones1)

  WnT = W_ih_n.T
  bn = (b_ih_n + b_hh_n).reshape(1, 256)
  WgT = W_gat.T
  M = jnp.repeat(jnp.eye(HEADS, dtype=F32), OUT, axis=0)
  As = att_src[0].reshape(HEADS * OUT, 1) * M
  Ad = att_dst[0].reshape(HEADS * OUT, 1) * M
  hn, cn, xl2, a_s, a_d, exs = _node_proj(
      x, WnT, bn, sums_p[0:N], sums_p[N:2 * N],
      cnt_p[0:N].reshape(N, 1), cnt_p[N:2 * N].reshape(N, 1), WgT, As, Ad)

  xlf = xl2.reshape(2 * N, 32)
  as0 = jnp.concatenate([a_s[:, 0], a_s[:, 2]])
  as1 = jnp.concatenate([a_s[:, 1], a_s[:, 3]])
  ad0 = jnp.concatenate([a_d[:, 0], a_d[:, 2]])
  ad1 = jnp.concatenate([a_d[:, 1], a_d[:, 3]])
  zacc = jnp.zeros((N, 32), F32)
  zden1 = jnp.zeros((N,), F32)
  acc, den0, den1 = _p2(src, dst, xlf, as0, as1, ad0, ad1, zacc, zden1)

  out = _finalize(acc[0:N], acc[N:2 * N],
                  den0[0:N].reshape(N, 1), den1[0:N].reshape(N, 1),
                  den0[N:2 * N].reshape(N, 1), den1[N:2 * N].reshape(N, 1),
                  xl2, exs, bias_gat.reshape(1, 16))

  return (out, hn[None], cn[None], h_e2[None], c_e2[None])
